# baseline (speedup 1.0000x reference)
import functools

import jax
import jax.numpy as jnp
from jax import lax
from jax.experimental import pallas as pl
from jax.experimental.pallas import tpu as pltpu
from jax.experimental.pallas import tpu_sc as plsc

B, T, D = 2, 2048, 768
E = 8
TOPK = 2
DFF = 4 * D
N = B * T
C = N // E
NW = 32
KD = 3072


def _sc_gather(table, idx, nrows=N):
    rpw = nrows // NW
    nch = max(rpw // 128, 1)
    cw = rpw // nch
    mesh = plsc.VectorSubcoreMesh(core_axis_name="c", subcore_axis_name="s")

    @functools.partial(
        pl.kernel,
        mesh=mesh,
        out_type=jax.ShapeDtypeStruct((nrows, D), jnp.float32),
        scratch_types=[
            pltpu.VMEM((nch, cw), jnp.int32),
            pltpu.VMEM((cw, D), jnp.float32),
            pltpu.SemaphoreType.DMA,
        ],
    )
    def k(table_hbm, idx_hbm, out_hbm, idx_v, rows_v, sem):
        wid = lax.axis_index("s") * 2 + lax.axis_index("c")
        base = wid * rpw
        pltpu.sync_copy(idx_hbm.at[wid], idx_v)
        for q in range(nch):
            pltpu.async_copy(table_hbm.at[idx_v.at[q]], rows_v, sem).wait()
            pltpu.sync_copy(rows_v,
                            out_hbm.at[pl.ds(base + q * cw, cw)])

    return k(table, idx.reshape(NW, nch, cw))


def _gelu_exact(h):
    return 0.5 * h * (1.0 + lax.erf(h / 1.4142135623730951))


def _tc_mlp(routed, fc_w, proj_w):
    grid = (E, DFF // KD)

    def body(r_ref, fc_ref, pj_ref, out_ref):
        kk = pl.program_id(1)
        a = r_ref[...]
        fw = fc_ref[0]
        h = lax.dot_general(a, fw, (((1,), (1,)), ((), ())),
                            preferred_element_type=jnp.float32)
        h = _gelu_exact(h)
        pw = pj_ref[0]
        contrib = lax.dot_general(h, pw, (((1,), (1,)), ((), ())),
                                  preferred_element_type=jnp.float32)

        @pl.when(kk == 0)
        def _():
            out_ref[...] = contrib

        @pl.when(kk > 0)
        def _():
            out_ref[...] += contrib

    return pl.pallas_call(
        body,
        grid=grid,
        in_specs=[
            pl.BlockSpec((C, D), lambda e, k: (e, 0)),
            pl.BlockSpec((1, KD, D), lambda e, k: (e, k, 0)),
            pl.BlockSpec((1, D, KD), lambda e, k: (e, 0, k)),
        ],
        out_specs=pl.BlockSpec((C, D), lambda e, k: (e, 0)),
        out_shape=jax.ShapeDtypeStruct((N, D), jnp.float32),
    )(routed, fc_w, proj_w)


def _tc_combine(gA, gB, wA, wB):
    def body(ga_ref, gb_ref, wa_ref, wb_ref, out_ref):
        out_ref[...] = ga_ref[...] * wa_ref[...] + gb_ref[...] * wb_ref[...]

    blk = 512
    return pl.pallas_call(
        body,
        grid=(N // blk,),
        in_specs=[
            pl.BlockSpec((blk, D), lambda i: (i, 0)),
            pl.BlockSpec((blk, D), lambda i: (i + N // blk, 0)),
            pl.BlockSpec((blk, 1), lambda i: (i, 0)),
            pl.BlockSpec((blk, 1), lambda i: (i, 0)),
        ],
        out_specs=pl.BlockSpec((blk, D), lambda i: (i, 0)),
        out_shape=jax.ShapeDtypeStruct((N, D), jnp.float32),
    )(gA, gB, wA, wB)


def kernel(x, gate_w, gate_b, fc_w, proj_w):
    flat = x.reshape(N, D)
    logits = flat @ gate_w.T + gate_b
    topv, topi = lax.top_k(logits, TOPK)
    rows = jnp.arange(N)[:, None]
    sparse = jnp.full_like(logits, -jnp.inf).at[rows, topi].set(topv)
    probs = jax.nn.softmax(sparse, axis=-1)
    pT = probs.T
    masked = jnp.where(pT > 0, pT, -jnp.inf)
    _, sel = lax.top_k(masked, C)
    tgt = sel.reshape(N).astype(jnp.int32)
    slotmap = jnp.full((E, N), -1, jnp.int32).at[
        jnp.arange(E)[:, None], sel].set(
        (jnp.arange(E)[:, None] * C + jnp.arange(C)[None, :]).astype(jnp.int32))
    tok = jnp.arange(N)
    sA = slotmap[topi[:, 0], tok]
    sB = slotmap[topi[:, 1], tok]
    pk = jnp.take_along_axis(probs, topi, axis=1)
    wA = jnp.where(sA >= 0, pk[:, 0], 0.0)[:, None]
    wB = jnp.where(sB >= 0, pk[:, 1], 0.0)[:, None]
    routed = _sc_gather(flat, tgt)
    eo = _tc_mlp(routed, fc_w, proj_w)
    catAB = jnp.concatenate([jnp.where(sA >= 0, sA, tok),
                             jnp.where(sB >= 0, sB, tok)]).astype(jnp.int32)
    gAB = _sc_gather(eo, catAB, nrows=2 * N)
    out = _tc_combine(gAB, gAB, wA, wB)
    return out.reshape(B, T, D)
